# baseline (speedup 1.0000x reference)
import functools

import jax
import jax.numpy as jnp
from jax import lax
from jax.experimental import pallas as pl
from jax.experimental.pallas import tpu as pltpu
from jax.experimental.pallas import tpu_sc as plsc

_N = 8388608
_ERASE_LEN = 1546698
_BEGIN = 3057263
_END = _BEGIN + _ERASE_LEN

_L = 16
_NC = 2
_NS = 16
_NW = _NC * _NS
_CH = 8192
_NBUF = 14
_NCHUNK = _N // _CH
_CPW = _NCHUNK // _NW
_ZB = 8192

_MB = _BEGIN // _CH
_ME = _END // _CH
_BA = (_BEGIN + _L - 1) // _L * _L
_EA = _END // _L * _L
_B_IN = _BEGIN - _MB * _CH
_BA_IN = _BA - _MB * _CH
_E_IN = _END - _ME * _CH
_EA_IN = _EA - _ME * _CH
_FE_LO = _MB + 1
_FE_HI = _ME

assert _N % _CH == 0 and _NCHUNK % _NW == 0 and _CH % _ZB == 0
assert _MB < _ME and _BA_IN % _L == 0 and _EA_IN % _L == 0
assert 0 < _BA_IN < _CH and _EA_IN + _L <= _CH


def _zero_fill(o_hbm, zbuf, sem, start, total):
    off = 0
    while off < total:
        n = min(_ZB, total - off)
        pltpu.make_async_copy(
            zbuf.at[pl.ds(0, n)], o_hbm.at[pl.ds(start + off, n)], sem
        ).start()
        off += n


def _sc_body(x_hbm, o_hbm, *rest):
    bufs = rest[:_NBUF]
    zbuf = rest[_NBUF]
    in_sems, out_sems, zsem = rest[_NBUF + 1], rest[_NBUF + 2], rest[_NBUF + 3]
    cid = lax.axis_index("c")
    sid = lax.axis_index("s")
    wid = sid * _NC + cid

    def _zb(i, carry):
        z = jnp.zeros((_L,), jnp.float32)
        base = i * (4 * _L)
        zbuf[pl.ds(base, _L)] = z
        zbuf[pl.ds(base + _L, _L)] = z
        zbuf[pl.ds(base + 2 * _L, _L)] = z
        zbuf[pl.ds(base + 3 * _L, _L)] = z
        return carry

    lax.fori_loop(0, _ZB // (4 * _L), _zb, 0)
    lane = lax.broadcasted_iota(jnp.int32, (_L,), 0)

    def chunk_idx(j):
        return wid + _NW * j

    def preds(j):
        m = chunk_idx(j)
        fully = (m >= _FE_LO) & (m < _FE_HI)
        is_b = m == _MB
        is_e = m == _ME
        plain = jnp.logical_not(fully | is_b | is_e)
        return fully, is_b, is_e, plain

    for j in range(_CPW):
        fully, is_b, is_e, _ = preds(j)
        lo = chunk_idx(j) * _CH

        @pl.when(fully)
        def _zero_chunk(lo=lo):
            _zero_fill(o_hbm, zbuf, zsem.at[0], lo, _CH)

        @pl.when(is_b)
        def _zero_b_tail(lo=lo):
            _zero_fill(o_hbm, zbuf, zsem.at[0], lo + _BA_IN, _CH - _BA_IN)

        @pl.when(is_e)
        def _zero_e_head(lo=lo):
            _zero_fill(o_hbm, zbuf, zsem.at[0], lo, _EA_IN)

    def in_desc(j):
        b = j % _NBUF
        return pltpu.make_async_copy(
            x_hbm.at[pl.ds(chunk_idx(j) * _CH, _CH)], bufs[b],
            in_sems.at[b])

    def start_in(j):
        fully, _, _, _ = preds(j)

        @pl.when(jnp.logical_not(fully))
        def _():
            in_desc(j).start()

    def drain_out(k):
        b = k % _NBUF
        fully, is_b, is_e, plain = preds(k)

        @pl.when(plain)
        def _():
            pltpu.make_async_copy(
                x_hbm.at[pl.ds(0, _CH)], bufs[b], out_sems.at[b]).wait()

        @pl.when(is_b)
        def _():
            pltpu.make_async_copy(
                x_hbm.at[pl.ds(0, _BA_IN)],
                bufs[b].at[pl.ds(0, _BA_IN)], out_sems.at[b]).wait()

        @pl.when(is_e)
        def _():
            pltpu.make_async_copy(
                x_hbm.at[pl.ds(0, _CH - _EA_IN)],
                bufs[b].at[pl.ds(0, _CH - _EA_IN)], out_sems.at[b]).wait()

    for k in range(min(_NBUF - 1, _CPW)):
        start_in(k)

    for j in range(_CPW):
        b = j % _NBUF
        buf = bufs[b]
        osem = out_sems.at[b]
        lo = chunk_idx(j) * _CH
        fully, is_b, is_e, plain = preds(j)

        @pl.when(jnp.logical_not(fully))
        def _wait_in():
            in_desc(j).wait()

        @pl.when(plain)
        def _store_plain():
            pltpu.make_async_copy(buf, o_hbm.at[pl.ds(lo, _CH)], osem).start()

        @pl.when(is_b)
        def _store_begin():
            base = _B_IN // _L * _L
            v = buf[pl.ds(base, _L)]
            buf[pl.ds(base, _L)] = jnp.where(
                lane >= _B_IN - base, jnp.float32(0.0), v)
            pltpu.make_async_copy(
                buf.at[pl.ds(0, _BA_IN)], o_hbm.at[pl.ds(lo, _BA_IN)], osem
            ).start()

        @pl.when(is_e)
        def _store_end():
            v = buf[pl.ds(_EA_IN, _L)]
            buf[pl.ds(_EA_IN, _L)] = jnp.where(
                lane < _E_IN - _EA_IN, jnp.float32(0.0), v)
            pltpu.make_async_copy(
                buf.at[pl.ds(_EA_IN, _CH - _EA_IN)],
                o_hbm.at[pl.ds(lo + _EA_IN, _CH - _EA_IN)], osem,
            ).start()

        nxt = j + _NBUF - 1
        if nxt < _CPW:
            if j >= 1:
                drain_out(j - 1)
            start_in(nxt)

    for j in range(max(_CPW - _NBUF, 0), _CPW):
        drain_out(j)
    for j in range(_CPW):
        fully, is_b, is_e, _ = preds(j)

        @pl.when(fully)
        def _():
            pltpu.make_async_copy(
                x_hbm.at[pl.ds(0, _CH)], bufs[0], zsem.at[0]).wait()

        @pl.when(is_b)
        def _():
            pltpu.make_async_copy(
                x_hbm.at[pl.ds(0, _CH - _BA_IN)],
                bufs[0].at[pl.ds(0, _CH - _BA_IN)], zsem.at[0]).wait()

        @pl.when(is_e)
        def _():
            pltpu.make_async_copy(
                x_hbm.at[pl.ds(0, _EA_IN)],
                bufs[0].at[pl.ds(0, _EA_IN)], zsem.at[0]).wait()


_sc_call = functools.partial(
    pl.kernel,
    out_type=jax.ShapeDtypeStruct((_N,), jnp.float32),
    mesh=plsc.VectorSubcoreMesh(core_axis_name="c", subcore_axis_name="s"),
    scratch_types=(
        [pltpu.VMEM((_CH,), jnp.float32) for _ in range(_NBUF)]
        + [
            pltpu.VMEM((_ZB,), jnp.float32),
            pltpu.SemaphoreType.DMA((_NBUF,)),
            pltpu.SemaphoreType.DMA((_NBUF,)),
            pltpu.SemaphoreType.DMA((1,)),
        ]
    ),
)(_sc_body)


def kernel(vector):
    return _sc_call(vector)


# speedup vs baseline: 1.0148x; 1.0148x over previous
import functools

import jax
import jax.numpy as jnp
from jax import lax
from jax.experimental import pallas as pl
from jax.experimental.pallas import tpu as pltpu
from jax.experimental.pallas import tpu_sc as plsc

_N = 8388608
_ERASE_LEN = 1546698
_BEGIN = 3057263
_END = _BEGIN + _ERASE_LEN

_L = 16
_NC = 2
_NS = 16
_NW = _NC * _NS
_CH = 8192
_NBUF = 14
_NCHUNK = _N // _CH
_CPW = _NCHUNK // _NW
_ZB = 8192

_MB = _BEGIN // _CH
_ME = _END // _CH
_BA = (_BEGIN + _L - 1) // _L * _L
_EA = _END // _L * _L
_B_IN = _BEGIN - _MB * _CH
_BA_IN = _BA - _MB * _CH
_E_IN = _END - _ME * _CH
_EA_IN = _EA - _ME * _CH
_FE_LO = _MB + 1
_FE_HI = _ME

assert _N % _CH == 0 and _NCHUNK % _NW == 0 and _CH % _ZB == 0
assert _MB < _ME and _BA_IN % _L == 0 and _EA_IN % _L == 0
assert 0 < _BA_IN < _CH and _EA_IN + _L <= _CH


def _zero_fill(o_hbm, zbuf, sem, start, total):
    off = 0
    while off < total:
        n = min(_ZB, total - off)
        pltpu.make_async_copy(
            zbuf.at[pl.ds(0, n)], o_hbm.at[pl.ds(start + off, n)], sem
        ).start()
        off += n


def _sc_body(x_hbm, o_hbm, *rest):
    bufs = rest[:_NBUF]
    zbuf = rest[_NBUF]
    in_sems, out_sems = rest[_NBUF + 1], rest[_NBUF + 2]
    cid = lax.axis_index("c")
    sid = lax.axis_index("s")
    wid = sid * _NC + cid

    def _zb(i, carry):
        z = jnp.zeros((_L,), jnp.float32)
        base = i * (4 * _L)
        zbuf[pl.ds(base, _L)] = z
        zbuf[pl.ds(base + _L, _L)] = z
        zbuf[pl.ds(base + 2 * _L, _L)] = z
        zbuf[pl.ds(base + 3 * _L, _L)] = z
        return carry

    lax.fori_loop(0, _ZB // (4 * _L), _zb, 0)
    lane = lax.broadcasted_iota(jnp.int32, (_L,), 0)

    def chunk_idx(j):
        return wid + _NW * j

    def not_full(j):
        m = chunk_idx(j)
        return jnp.logical_not((m >= _FE_LO) & (m < _FE_HI))

    def in_desc(j):
        b = j % _NBUF
        return pltpu.make_async_copy(
            x_hbm.at[pl.ds(chunk_idx(j) * _CH, _CH)], bufs[b],
            in_sems.at[b])

    def start_in(j):
        @pl.when(not_full(j))
        def _():
            in_desc(j).start()

    def drain_out(b):
        pltpu.make_async_copy(
            x_hbm.at[pl.ds(0, _CH)], bufs[b], out_sems.at[b]
        ).wait()

    for k in range(min(_NBUF - 1, _CPW)):
        start_in(k)

    for j in range(_CPW):
        b = j % _NBUF
        buf = bufs[b]
        osem = out_sems.at[b]
        m = chunk_idx(j)
        lo = m * _CH
        fully_erased = (m >= _FE_LO) & (m < _FE_HI)
        is_b = m == _MB
        is_e = m == _ME
        plain = jnp.logical_not(fully_erased | is_b | is_e)

        @pl.when(jnp.logical_not(fully_erased))
        def _wait_in():
            in_desc(j).wait()

        @pl.when(plain)
        def _store_plain():
            pltpu.make_async_copy(buf, o_hbm.at[pl.ds(lo, _CH)], osem).start()

        @pl.when(fully_erased)
        def _store_zeros():
            _zero_fill(o_hbm, zbuf, osem, lo, _CH)

        @pl.when(is_b)
        def _store_begin():
            base = _B_IN // _L * _L
            v = buf[pl.ds(base, _L)]
            buf[pl.ds(base, _L)] = jnp.where(
                lane >= _B_IN - base, jnp.float32(0.0), v)
            pltpu.make_async_copy(
                buf.at[pl.ds(0, _BA_IN)], o_hbm.at[pl.ds(lo, _BA_IN)], osem
            ).start()
            _zero_fill(o_hbm, zbuf, osem, lo + _BA_IN, _CH - _BA_IN)

        @pl.when(is_e)
        def _store_end():
            v = buf[pl.ds(_EA_IN, _L)]
            buf[pl.ds(_EA_IN, _L)] = jnp.where(
                lane < _E_IN - _EA_IN, jnp.float32(0.0), v)
            _zero_fill(o_hbm, zbuf, osem, lo, _EA_IN)
            pltpu.make_async_copy(
                buf.at[pl.ds(_EA_IN, _CH - _EA_IN)],
                o_hbm.at[pl.ds(lo + _EA_IN, _CH - _EA_IN)], osem,
            ).start()

        nxt = j + _NBUF - 1
        if nxt < _CPW:
            if j >= 1:
                drain_out((j - 1) % _NBUF)
            start_in(nxt)

    for j in range(max(_CPW - _NBUF, 0), _CPW):
        drain_out(j % _NBUF)


_sc_call = functools.partial(
    pl.kernel,
    out_type=jax.ShapeDtypeStruct((_N,), jnp.float32),
    mesh=plsc.VectorSubcoreMesh(core_axis_name="c", subcore_axis_name="s"),
    scratch_types=(
        [pltpu.VMEM((_CH,), jnp.float32) for _ in range(_NBUF)]
        + [
            pltpu.VMEM((_ZB,), jnp.float32),
            pltpu.SemaphoreType.DMA((_NBUF,)),
            pltpu.SemaphoreType.DMA((_NBUF,)),
        ]
    ),
)(_sc_body)


def kernel(vector):
    return _sc_call(vector)
